# Initial kernel scaffold; baseline (speedup 1.0000x reference)
import jax
import jax.numpy as jnp
from jax.experimental import pallas as pl


def kernel(x, edge_index, x2, temp, W1, b1, W2, b2, Wp, bp, W3, W4):
    raise NotImplementedError("write your pallas kernel here")


# speedup vs baseline: 30.7920x; 30.7920x over previous
import functools

import jax
import jax.numpy as jnp
from jax import lax
from jax.experimental import pallas as pl
from jax.experimental.pallas import tpu as pltpu
from jax.experimental.pallas import tpu_sc as plsc

N = 10000
D = 128
C = 128
E = 320000
NC = 2
NS = 16
L = 16

MAX_LOGSTD = 10.0
SCALING = 1.8
SC_GAIN = 0.8

_MESH = dict(core_axis_name="c", subcore_axis_name="s")


_K2_CH = 2000
_K2_PER_TILE = E // (NC * NS)


def _k2_body(dst_hbm, ones_hbm, zeros_hbm, cnt0_hbm, cnt1_hbm,
             idx_v, ones_v, cnt_sh):
    cid = lax.axis_index("c")
    sid = lax.axis_index("s")

    @pl.when(sid == 0)
    def _():
        pltpu.sync_copy(zeros_hbm, cnt_sh)

    pltpu.sync_copy(ones_hbm, ones_v)
    plsc.subcore_barrier()

    base = (cid * NS + sid) * _K2_PER_TILE

    def chunk(ci, carry):
        off = base + ci * _K2_CH
        pltpu.sync_copy(dst_hbm.at[pl.ds(off, _K2_CH)], idx_v)
        pltpu.sync_copy(ones_v, cnt_sh.at[idx_v], add=True)
        return carry

    lax.fori_loop(0, _K2_PER_TILE // _K2_CH, chunk, 0)
    plsc.subcore_barrier()

    @pl.when(sid == 0)
    def _():
        @pl.when(cid == 0)
        def _():
            pltpu.sync_copy(cnt_sh, cnt0_hbm)

        @pl.when(cid == 1)
        def _():
            pltpu.sync_copy(cnt_sh, cnt1_hbm)


def _k2_call(dst, ones, zeros):
    f = pl.kernel(
        _k2_body,
        out_type=(jax.ShapeDtypeStruct((N,), jnp.float32),
                  jax.ShapeDtypeStruct((N,), jnp.float32)),
        mesh=plsc.VectorSubcoreMesh(**_MESH),
        scratch_types=[
            pltpu.VMEM((_K2_CH,), jnp.int32),
            pltpu.VMEM((_K2_CH,), jnp.float32),
            pltpu.VMEM_SHARED((N,), jnp.float32),
        ],
    )
    return f(dst, ones, zeros)


_K13_BN = 512
_K13_G = pl.cdiv(N, _K13_BN)
_DOTF32 = (((1,), (1,)), ((), ()))


def _k13_body(x_ref, w1_ref, w2_ref, wp_ref, b1_ref, b2_ref, c0_ref, c1_ref,
              fa_ref, fb_ref, dinv_ref):
    xb = x_ref[...]
    deg = c0_ref[...] + c1_ref[...] + 1.0
    dinv = 1.0 / jnp.sqrt(jnp.maximum(deg, 1e-12))
    a1 = lax.dot_general(xb, w1_ref[...], _DOTF32,
                         preferred_element_type=jnp.float32) + b1_ref[...]
    fa_ref[...] = lax.dot_general(a1, wp_ref[...], _DOTF32,
                                  preferred_element_type=jnp.float32) * dinv
    t2 = lax.dot_general(xb, w2_ref[...], _DOTF32,
                         preferred_element_type=jnp.float32) + b2_ref[...]
    nrm = jnp.sqrt(jnp.sum(t2 * t2, axis=1, keepdims=True))
    mm = t2 / jnp.maximum(nrm, 1e-12) * SCALING
    fb_ref[...] = lax.dot_general(mm, wp_ref[...], _DOTF32,
                                  preferred_element_type=jnp.float32) * dinv
    dinv_ref[...] = dinv


def _k13_call(x, W1, W2, Wp, b1, b2, cnt0, cnt1):
    row = pl.BlockSpec((_K13_BN, 128), lambda i: (i, 0))
    full = pl.BlockSpec((128, 128), lambda i: (0, 0))
    bias = pl.BlockSpec((1, 128), lambda i: (0, 0))
    cblk = pl.BlockSpec((_K13_BN, 1), lambda i: (i, 0))
    return pl.pallas_call(
        _k13_body,
        grid=(_K13_G,),
        in_specs=[row, full, full, full, bias, bias, cblk, cblk],
        out_specs=[row, row, pl.BlockSpec((_K13_BN, 1), lambda i: (i, 0))],
        out_shape=[jax.ShapeDtypeStruct((N, C), jnp.float32),
                   jax.ShapeDtypeStruct((N, C), jnp.float32),
                   jax.ShapeDtypeStruct((N, 1), jnp.float32)],
    )(x, W1, W2, Wp, b1.reshape(1, C), b2.reshape(1, C),
      cnt0.reshape(N, 1), cnt1.reshape(N, 1))


_K4_CH = 160
_K4_PER_TILE = E // NS
_K4_NCH = _K4_PER_TILE // _K4_CH
_ROW_A = 624
_ROW_LAST = N - _ROW_A * (NS - 1)


def _rowshard(sid, body):
    start = pl.multiple_of(sid * _ROW_A, 8)

    @pl.when(sid < NS - 1)
    def _():
        body(start, _ROW_A)

    @pl.when(sid == NS - 1)
    def _():
        body(_ROW_A * (NS - 1), _ROW_LAST)


def _k4_body(fa_hbm, fb_hbm, src_hbm, dst_hbm, aggA_hbm, aggB_hbm,
             idxS_a, idxD_a, idxS_b, idxD_b, rows_a, rows_b, agg_sh,
             semA, semB):
    cid = lax.axis_index("c")
    sid = lax.axis_index("s")

    def init_from(feat):
        _rowshard(sid, lambda s, n: pltpu.sync_copy(
            feat.at[pl.ds(s, n)], agg_sh.at[pl.ds(s, n)]))

    @pl.when(cid == 0)
    def _():
        init_from(fa_hbm)

    @pl.when(cid == 1)
    def _():
        init_from(fb_hbm)

    plsc.subcore_barrier()

    def sweep(feat):
        base = sid * _K4_PER_TILE

        def start(ci, iS, iD, sem, rows):
            off = base + ci * _K4_CH
            pltpu.sync_copy(src_hbm.at[pl.ds(off, _K4_CH)], iS)
            pltpu.sync_copy(dst_hbm.at[pl.ds(off, _K4_CH)], iD)
            pltpu.async_copy(feat.at[iS], rows, sem)

        def drain_scatter(iS, iD, sem, rows):
            pltpu.make_async_copy(feat.at[iS], rows, sem).wait()
            pltpu.sync_copy(rows, agg_sh.at[iD], add=True)

        start(0, idxS_a, idxD_a, semA, rows_a)

        def pair(i, carry):
            start(2 * i + 1, idxS_b, idxD_b, semB, rows_b)
            drain_scatter(idxS_a, idxD_a, semA, rows_a)

            @pl.when(2 * i + 2 < _K4_NCH)
            def _():
                start(2 * i + 2, idxS_a, idxD_a, semA, rows_a)

            drain_scatter(idxS_b, idxD_b, semB, rows_b)
            return carry

        lax.fori_loop(0, _K4_NCH // 2, pair, 0)
        if _K4_NCH % 2 == 1:
            drain_scatter(idxS_a, idxD_a, semA, rows_a)

    @pl.when(cid == 0)
    def _():
        sweep(fa_hbm)

    @pl.when(cid == 1)
    def _():
        sweep(fb_hbm)

    plsc.subcore_barrier()

    @pl.when(cid == 0)
    def _():
        _rowshard(sid, lambda s, n: pltpu.sync_copy(
            agg_sh.at[pl.ds(s, n)], aggA_hbm.at[pl.ds(s, n)]))

    @pl.when(cid == 1)
    def _():
        _rowshard(sid, lambda s, n: pltpu.sync_copy(
            agg_sh.at[pl.ds(s, n)], aggB_hbm.at[pl.ds(s, n)]))


def _k4_call(fa, fb, src, dst):
    f = pl.kernel(
        _k4_body,
        out_type=(jax.ShapeDtypeStruct((N, C), jnp.float32),
                  jax.ShapeDtypeStruct((N, C), jnp.float32)),
        mesh=plsc.VectorSubcoreMesh(**_MESH),
        scratch_types=[
            pltpu.VMEM((_K4_CH,), jnp.int32),
            pltpu.VMEM((_K4_CH,), jnp.int32),
            pltpu.VMEM((_K4_CH,), jnp.int32),
            pltpu.VMEM((_K4_CH,), jnp.int32),
            pltpu.VMEM((_K4_CH, C), jnp.float32),
            pltpu.VMEM((_K4_CH, C), jnp.float32),
            pltpu.VMEM_SHARED((N, C), jnp.float32),
            pltpu.SemaphoreType.DMA,
            pltpu.SemaphoreType.DMA,
        ],
    )
    return f(fa, fb, src, dst)


_K6_BN = 200
_K6_G = (N // 2) // _K6_BN
_K6_EB = (E // 2) // _K6_G


def _k6_body(x2_ref, w_ref, e20_ref, u_ref, z2_ref, g_ref):
    t = lax.dot_general(x2_ref[...], w_ref[...], _DOTF32,
                        preferred_element_type=jnp.float32)
    t0, t1, ls = t[:, 0:1], t[:, 1:2], t[:, 2:3]
    n2 = jnp.sqrt(t0 * t0 + t1 * t1)
    mu20 = t0 / jnp.maximum(n2, 1e-12) * SC_GAIN
    z2_ref[...] = mu20 + e20_ref[...] * jnp.exp(jnp.minimum(ls, MAX_LOGSTD))
    g_ref[...] = -jnp.log(-jnp.log(u_ref[...]))


def _k6_call(x2, w34, e20, uT, half):
    rb = half * _K6_G
    return pl.pallas_call(
        _k6_body,
        grid=(_K6_G,),
        in_specs=[pl.BlockSpec((_K6_BN, N), lambda i: (rb + i, 0)),
                  pl.BlockSpec((4, N), lambda i: (0, 0)),
                  pl.BlockSpec((_K6_BN, 1), lambda i: (rb + i, 0)),
                  pl.BlockSpec((2, _K6_EB), lambda i: (0, rb + i))],
        out_specs=[pl.BlockSpec((_K6_BN, 1), lambda i: (i, 0)),
                   pl.BlockSpec((2, _K6_EB), lambda i: (0, i))],
        out_shape=[jax.ShapeDtypeStruct((N // 2, 1), jnp.float32),
                   jax.ShapeDtypeStruct((2, E // 2), jnp.float32)],
    )(x2, w34, e20, uT)


def _k5_body(aggA_ref, aggB_ref, dinv_ref, bp_ref, eps_ref, z1_ref):
    dinv = dinv_ref[...]
    logstd1 = jnp.minimum(dinv * aggA_ref[...] + bp_ref[...], MAX_LOGSTD)
    mu1 = dinv * aggB_ref[...] + bp_ref[...]
    z1_ref[...] = mu1 + eps_ref[...] * jnp.exp(logstd1)


def _k5_call(aggA, aggB, dinv, bp, eps1):
    row = pl.BlockSpec((_K13_BN, 128), lambda i: (i, 0))
    return pl.pallas_call(
        _k5_body,
        grid=(_K13_G,),
        in_specs=[row, row, pl.BlockSpec((_K13_BN, 1), lambda i: (i, 0)),
                  pl.BlockSpec((1, 128), lambda i: (0, 0)), row],
        out_specs=row,
        out_shape=jax.ShapeDtypeStruct((N, C), jnp.float32),
    )(aggA, aggB, dinv, bp.reshape(1, C), eps1)


_K7_CH = 200
_K7_PAD = _K7_CH + L
_K7_PER_TILE = E // (NC * NS)
_K7_NCH = _K7_PER_TILE // _K7_CH
_K7_GROUPS = (_K7_CH + L - 1) // L
_K7_SB = _K7_CH * L + L


def _k7_body(z1_hbm, src_hbm, dst_hbm, vf_hbm,
             iSa, iDa, iSb, iDb,
             rSa, rDa, rSb, rDb, sbuf, outv,
             sSa, sDa, sSb, sDb):
    cid = lax.axis_index("c")
    sid = lax.axis_index("s")
    wid = sid * NC + cid
    base = wid * _K7_PER_TILE

    lanes = lax.iota(jnp.int32, L)
    bufA = (iSa, iDa, rSa, rDa, sSa, sDa)
    bufB = (iSb, iDb, rSb, rDb, sSb, sDb)

    def start(ci, buf):
        iS, iD, rS, rD, semS, semD = buf
        off = base + ci * _K7_CH
        pltpu.sync_copy(src_hbm.at[pl.ds(off, _K7_CH)], iS)
        pltpu.sync_copy(dst_hbm.at[pl.ds(off, _K7_CH)], iD)
        pltpu.async_copy(z1_hbm.at[iS], rS, semS)
        pltpu.async_copy(z1_hbm.at[iD], rD, semD)

    def compute(ci, buf):
        iS, iD, rS, rD, semS, semD = buf
        off = base + ci * _K7_CH
        pltpu.make_async_copy(z1_hbm.at[iS], rS, semS).wait()
        pltpu.make_async_copy(z1_hbm.at[iD], rD, semD).wait()

        @plsc.parallel_loop(0, _K7_CH, unroll=2)
        def _(e):
            accs = [rS[e, pl.ds(c * L, L)] * rD[e, pl.ds(c * L, L)]
                    for c in range(4)]
            for c in range(4, C // L):
                accs[c % 4] = accs[c % 4] + \
                    rS[e, pl.ds(c * L, L)] * rD[e, pl.ds(c * L, L)]
            s = (accs[0] + accs[1]) + (accs[2] + accs[3])
            plsc.store_scatter(sbuf, [lanes * _K7_CH + e], s)

        def group(gi, gcarry):
            sl = pl.ds(gi * L, L)
            vs = [sbuf[pl.ds(l * _K7_CH + gi * L, L)] for l in range(L)]
            t = [vs[2 * i] + vs[2 * i + 1] for i in range(8)]
            t = [t[2 * i] + t[2 * i + 1] for i in range(4)]
            outv[sl] = (t[0] + t[1]) + (t[2] + t[3])
            return gcarry

        lax.fori_loop(0, _K7_GROUPS, group, 0)
        pltpu.sync_copy(outv.at[pl.ds(0, _K7_CH)], vf_hbm.at[pl.ds(off, _K7_CH)])

    start(0, bufA)

    def pair(i, carry):
        start(2 * i + 1, bufB)
        compute(2 * i, bufA)

        @pl.when(2 * i + 2 < _K7_NCH)
        def _():
            start(2 * i + 2, bufA)

        compute(2 * i + 1, bufB)
        return carry

    lax.fori_loop(0, _K7_NCH // 2, pair, 0)


def _k7_call(z1, src, dst):
    f = pl.kernel(
        _k7_body,
        out_type=jax.ShapeDtypeStruct((E,), jnp.float32),
        mesh=plsc.VectorSubcoreMesh(**_MESH),
        compiler_params=pltpu.CompilerParams(needs_layout_passes=False),
        scratch_types=[
            pltpu.VMEM((_K7_CH,), jnp.int32),
            pltpu.VMEM((_K7_CH,), jnp.int32),
            pltpu.VMEM((_K7_CH,), jnp.int32),
            pltpu.VMEM((_K7_CH,), jnp.int32),
            pltpu.VMEM((_K7_CH, C), jnp.float32),
            pltpu.VMEM((_K7_CH, C), jnp.float32),
            pltpu.VMEM((_K7_CH, C), jnp.float32),
            pltpu.VMEM((_K7_CH, C), jnp.float32),
            pltpu.VMEM((_K7_SB,), jnp.float32),
            pltpu.VMEM((_K7_PAD,), jnp.float32),
            pltpu.SemaphoreType.DMA,
            pltpu.SemaphoreType.DMA,
            pltpu.SemaphoreType.DMA,
            pltpu.SemaphoreType.DMA,
        ],
    )
    return f(z1, src, dst)


_K7B_CH = _K7_PER_TILE


def _k7b_body(z2_hbm, src_hbm, dst_hbm, g0_hbm, g1_hbm, vf_hbm, out_hbm,
              iS, iD, g0v, g1v, vfv, outv, z2loc):
    cid = lax.axis_index("c")
    sid = lax.axis_index("s")
    wid = sid * NC + cid
    off = wid * _K7B_CH

    pltpu.sync_copy(z2_hbm, z2loc)
    pltpu.sync_copy(src_hbm.at[pl.ds(off, _K7B_CH)], iS)
    pltpu.sync_copy(dst_hbm.at[pl.ds(off, _K7B_CH)], iD)
    pltpu.sync_copy(g0_hbm.at[pl.ds(off, _K7B_CH)], g0v)
    pltpu.sync_copy(g1_hbm.at[pl.ds(off, _K7B_CH)], g1v)
    pltpu.sync_copy(vf_hbm.at[pl.ds(off, _K7B_CH)], vfv)

    def group(gi, carry):
        sl = pl.ds(gi * L, L)
        vf = vfv[sl]
        keep = (vf + g0v[sl]) >= g1v[sl]
        sig_f = 1.0 / (1.0 + jnp.exp(-vf))
        vn = plsc.load_gather(z2loc, [iS[sl]]) + \
             plsc.load_gather(z2loc, [iD[sl]])
        sig_n = 1.0 / (1.0 + jnp.exp(-vn))
        outv[sl] = jnp.where(keep, sig_f, sig_n)
        return carry

    lax.fori_loop(0, _K7B_CH // L, group, 0)
    pltpu.sync_copy(outv, out_hbm.at[pl.ds(off, _K7B_CH)])


def _k7b_call(z2v, src, dst, g0, g1, vf):
    f = pl.kernel(
        _k7b_body,
        out_type=jax.ShapeDtypeStruct((E,), jnp.float32),
        mesh=plsc.VectorSubcoreMesh(**_MESH),
        compiler_params=pltpu.CompilerParams(needs_layout_passes=False),
        scratch_types=[
            pltpu.VMEM((_K7B_CH,), jnp.int32),
            pltpu.VMEM((_K7B_CH,), jnp.int32),
            pltpu.VMEM((_K7B_CH,), jnp.float32),
            pltpu.VMEM((_K7B_CH,), jnp.float32),
            pltpu.VMEM((_K7B_CH,), jnp.float32),
            pltpu.VMEM((_K7B_CH,), jnp.float32),
            pltpu.VMEM((N,), jnp.float32),
        ],
    )
    return f(z2v, src, dst, g0, g1, vf)


def kernel(x, edge_index, x2, temp, W1, b1, W2, b2, Wp, bp, W3, W4):
    key = jax.random.key(42)
    k1, k2, k3 = jax.random.split(key, 3)

    src = edge_index[0]
    dst = edge_index[1]

    ones = jnp.ones((_K2_CH,), jnp.float32)
    zeros = jnp.zeros((N,), jnp.float32)
    cnt0, cnt1 = _k2_call(dst, ones, zeros)

    fa, fb, dinv = _k13_call(x, W1, W2, Wp, b1, b2, cnt0, cnt1)
    aggA, aggB = _k4_call(fa, fb, src, dst)

    eps2 = jax.random.normal(k2, (N, 2), dtype=jnp.float32)
    u = jax.random.uniform(k3, (E, 2), minval=1e-10, maxval=1.0)
    w34 = jnp.concatenate([W4, W3], axis=0)
    e20 = eps2[:, 0:1]
    uT = u.T
    z2a, gTa = _k6_call(x2, w34, e20, uT, 0)
    eps1 = jax.random.normal(k1, (N, C), dtype=jnp.float32)

    aggA, aggB, z2a, gTa = lax.optimization_barrier((aggA, aggB, z2a, gTa))
    z1 = _k5_call(aggA, aggB, dinv, bp, eps1)

    vf = _k7_call(z1, src, dst)
    z2b, gTb = _k6_call(x2, w34, e20, uT, 1)

    z2c = jnp.concatenate([z2a, z2b], axis=0)
    g0 = jnp.concatenate([gTa[0], gTb[0]], axis=0)
    g1 = jnp.concatenate([gTa[1], gTb[1]], axis=0)
    res = _k7b_call(z2c.reshape(N), src, dst, g0, g1, vf)
    return res
